# baseline (speedup 1.0000x reference)
import functools

import jax
import jax.numpy as jnp
from jax import lax
from jax.experimental import pallas as pl
from jax.experimental.pallas import tpu as pltpu
from jax.experimental.pallas import tpu_sc as plsc

EMBED_DIM = 32
BATCH = 16384

_info = plsc.get_sparse_core_info()
_NC, _NS = _info.num_cores, _info.num_subcores
_NW = _NC * _NS
_B_PER_W = BATCH // _NW
_CHUNK = 128
_NCH = _B_PER_W // _CHUNK

_mesh = plsc.VectorSubcoreMesh(core_axis_name="c", subcore_axis_name="s")


@functools.partial(
    pl.kernel,
    mesh=_mesh,
    out_type=jax.ShapeDtypeStruct((_NW, _NCH, _CHUNK, EMBED_DIM), jnp.float32),
    scratch_types=[
        pltpu.VMEM((_NCH, _CHUNK), jnp.int32),
        pltpu.VMEM((_NCH, _CHUNK, EMBED_DIM), jnp.float32),
        pltpu.SemaphoreType.DMA,
    ],
    compiler_params=pltpu.CompilerParams(use_tc_tiling_on_sc=False),
)
def _gather_rows(idx_hbm, table_hbm, out_hbm, idx_v, rows_v, sem):
    wid = lax.axis_index("s") * _NC + lax.axis_index("c")
    pltpu.sync_copy(idx_hbm.at[wid], idx_v)
    copies = [
        pltpu.async_copy(table_hbm.at[idx_v.at[j]], rows_v.at[j], sem)
        for j in range(_NCH)
    ]
    for c in copies:
        c.wait()
    pltpu.sync_copy(rows_v, out_hbm.at[wid])


def kernel(task_id, table):
    idx = task_id.astype(jnp.int32).reshape(_NW, _NCH, _CHUNK)
    out = _gather_rows(idx, table)
    return out.reshape(BATCH, EMBED_DIM)


# speedup vs baseline: 3.2251x; 3.2251x over previous
import functools

import jax
import jax.numpy as jnp
from jax import lax
from jax.experimental import pallas as pl
from jax.experimental.pallas import tpu as pltpu
from jax.experimental.pallas import tpu_sc as plsc

EMBED_DIM = 32
BATCH = 16384
IDX_PAD = 32

_info = plsc.get_sparse_core_info()
_NC, _NS = _info.num_cores, _info.num_subcores
_NW = _NC * _NS
_B_PER_W = BATCH // _NW
_BLK = 128
_NBLK = _B_PER_W // _BLK
_RING = 4

_mesh = plsc.VectorSubcoreMesh(core_axis_name="c", subcore_axis_name="s")


@functools.partial(
    pl.kernel,
    mesh=_mesh,
    out_type=jax.ShapeDtypeStruct((EMBED_DIM, BATCH), jnp.float32),
    scratch_types=[
        pltpu.VMEM((_B_PER_W + IDX_PAD,), jnp.int32),
        pltpu.VMEM((EMBED_DIM, 128), jnp.float32),
        pltpu.VMEM((EMBED_DIM, 128), jnp.float32),
        pltpu.VMEM((EMBED_DIM, 128), jnp.float32),
        pltpu.VMEM((EMBED_DIM, 128), jnp.float32),
        pltpu.VMEM((EMBED_DIM, _BLK), jnp.float32),
        pltpu.SemaphoreType.DMA,
        pltpu.SemaphoreType.DMA,
        pltpu.SemaphoreType.DMA,
        pltpu.SemaphoreType.DMA,
        pltpu.SemaphoreType.DMA,
    ],
    compiler_params=pltpu.CompilerParams(
        use_tc_tiling_on_sc=True, needs_layout_passes=False
    ),
)
def _gather_windows(
    idx_hbm, tableT_hbm, outT_hbm,
    idx_v, win0, win1, win2, win3, stage_v,
    sem0, sem1, sem2, sem3, sem_i,
):
    wins = (win0, win1, win2, win3)
    sems = (sem0, sem1, sem2, sem3)
    wid = lax.axis_index("s") * _NC + lax.axis_index("c")
    base = wid * _B_PER_W
    pltpu.async_copy(
        idx_hbm.at[pl.ds(base, _B_PER_W)],
        idx_v.at[pl.ds(0, _B_PER_W)],
        sem_i,
    ).wait()
    zeros16 = jnp.zeros((16,), jnp.int32)
    idx_v[pl.ds(_B_PER_W, 16)] = zeros16
    idx_v[pl.ds(_B_PER_W + 16, 16)] = zeros16

    iota16 = lax.iota(jnp.int32, 16)

    def fire(q_idx_scalar, slot):
        jb = lax.shift_right_logical(q_idx_scalar, 7)
        off = pl.multiple_of(jb * 128, 128)
        pltpu.async_copy(
            tableT_hbm.at[:, pl.ds(off, 128)], wins[slot], sems[slot]
        )

    def extract(q_idx_scalar, qcol, slot):
        l = jnp.full((16,), q_idx_scalar & 127, jnp.int32)
        qv = jnp.full((16,), qcol, jnp.int32)
        lo = plsc.load_gather(wins[slot], [iota16, l])
        hi = plsc.load_gather(wins[slot], [iota16 + 16, l])
        plsc.store_scatter(stage_v, [iota16, qv], lo)
        plsc.store_scatter(stage_v, [iota16 + 16, qv], hi)

    def wait_slot(slot):
        pltpu.make_async_copy(
            tableT_hbm.at[:, pl.ds(0, 128)], wins[slot], sems[slot]
        ).wait()

    for blk in range(_NBLK):
        bbase = blk * _BLK
        v0 = idx_v[pl.ds(bbase, 16)]
        for j in range(_RING):
            fire(v0[j], j)

        def group_body(g, carry):
            v = idx_v[pl.ds(bbase + g * 16, 16)]
            vn = idx_v[pl.ds(bbase + g * 16 + 16, 16)]
            for j in range(16):
                qloc = g * 16 + j
                slot = j % _RING
                wait_slot(slot)
                extract(v[j], qloc, slot)
                nxt = v[j + 4] if j < 12 else vn[j - 12]
                if j < 12:
                    fire(nxt, slot)
                else:
                    @pl.when(g < (_BLK // 16) - 1)
                    def _():
                        fire(nxt, slot)
            return carry

        lax.fori_loop(0, _BLK // 16, group_body, 0)
        pltpu.sync_copy(
            stage_v,
            outT_hbm.at[:, pl.ds(pl.multiple_of(base + bbase, 128), _BLK)],
        )


def kernel(task_id, table):
    idx = task_id.astype(jnp.int32)
    outT = _gather_windows(idx, table.T)
    return outT.T


# speedup vs baseline: 4.0815x; 1.2655x over previous
import functools

import jax
import jax.numpy as jnp
from jax import lax
from jax.experimental import pallas as pl
from jax.experimental.pallas import tpu as pltpu
from jax.experimental.pallas import tpu_sc as plsc

EMBED_DIM = 32
BATCH = 16384
IDX_PAD = 32

_info = plsc.get_sparse_core_info()
_NC, _NS = _info.num_cores, _info.num_subcores
_NW = _NC * _NS
_B_PER_W = BATCH // _NW
_BLK = 128
_NBLK = _B_PER_W // _BLK
_RING = 8

_mesh = plsc.VectorSubcoreMesh(core_axis_name="c", subcore_axis_name="s")


@functools.partial(
    pl.kernel,
    mesh=_mesh,
    out_type=jax.ShapeDtypeStruct((EMBED_DIM, BATCH), jnp.float32),
    scratch_types=[
        pltpu.VMEM((_B_PER_W + IDX_PAD,), jnp.int32),
        *([pltpu.VMEM((EMBED_DIM, 128), jnp.float32)] * 8),
        pltpu.VMEM((EMBED_DIM, _BLK), jnp.float32),
        *([pltpu.SemaphoreType.DMA] * 9),
    ],
    compiler_params=pltpu.CompilerParams(
        use_tc_tiling_on_sc=True, needs_layout_passes=False
    ),
)
def _gather_windows(
    idx_hbm, tableT_hbm, outT_hbm,
    idx_v, w0, w1, w2, w3, w4, w5, w6, w7, stage_v,
    s0, s1, s2, s3, s4, s5, s6, s7, sem_i,
):
    wins = (w0, w1, w2, w3, w4, w5, w6, w7)
    sems = (s0, s1, s2, s3, s4, s5, s6, s7)
    wid = lax.axis_index("s") * _NC + lax.axis_index("c")
    base = wid * _B_PER_W
    pltpu.async_copy(
        idx_hbm.at[pl.ds(base, _B_PER_W)],
        idx_v.at[pl.ds(0, _B_PER_W)],
        sem_i,
    ).wait()
    zeros16 = jnp.zeros((16,), jnp.int32)
    idx_v[pl.ds(_B_PER_W, 16)] = zeros16
    idx_v[pl.ds(_B_PER_W + 16, 16)] = zeros16

    iota16 = lax.iota(jnp.int32, 16)

    def fire(q_idx_scalar, slot):
        jb = lax.shift_right_logical(q_idx_scalar, 7)
        off = pl.multiple_of(jb * 128, 128)
        pltpu.async_copy(
            tableT_hbm.at[:, pl.ds(off, 128)], wins[slot], sems[slot]
        )

    def extract(q_idx_scalar, qcol, slot):
        l = jnp.full((16,), q_idx_scalar & 127, jnp.int32)
        qv = jnp.full((16,), qcol, jnp.int32)
        lo = plsc.load_gather(wins[slot], [iota16, l])
        hi = plsc.load_gather(wins[slot], [iota16 + 16, l])
        plsc.store_scatter(stage_v, [iota16, qv], lo)
        plsc.store_scatter(stage_v, [iota16 + 16, qv], hi)

    def wait_slot(slot):
        pltpu.make_async_copy(
            tableT_hbm.at[:, pl.ds(0, 128)], wins[slot], sems[slot]
        ).wait()

    for blk in range(_NBLK):
        bbase = blk * _BLK
        v0 = idx_v[pl.ds(bbase, 16)]
        for j in range(_RING):
            fire(v0[j], j)

        def group_body(g, carry):
            v = idx_v[pl.ds(bbase + g * 16, 16)]
            vn = idx_v[pl.ds(bbase + g * 16 + 16, 16)]
            for j in range(16):
                qloc = g * 16 + j
                slot = j % _RING
                wait_slot(slot)
                extract(v[j], qloc, slot)
                nxt = v[j + _RING] if j < 16 - _RING else vn[j - (16 - _RING)]
                if j < 16 - _RING:
                    fire(nxt, slot)
                else:
                    @pl.when(g < (_BLK // 16) - 1)
                    def _():
                        fire(nxt, slot)
            return carry

        lax.fori_loop(0, _BLK // 16, group_body, 0)
        pltpu.sync_copy(
            stage_v,
            outT_hbm.at[:, pl.ds(pl.multiple_of(base + bbase, 128), _BLK)],
        )


def kernel(task_id, table):
    idx = task_id.astype(jnp.int32)
    outT = _gather_windows(idx, table.T)
    return outT.T
